# baseline (speedup 1.0000x reference)
import functools

import jax
import jax.numpy as jnp
from jax import lax
from jax.experimental import pallas as pl
from jax.experimental.pallas import tpu as pltpu
from jax.experimental.pallas import tpu_sc as plsc

NC = 2
NS = 16
NW = NC * NS
K = 128
D = 32
CHUNKS = 100

_mesh = plsc.VectorSubcoreMesh(core_axis_name="c", subcore_axis_name="s")


@functools.partial(
    pl.kernel,
    mesh=_mesh,
    compiler_params=pltpu.CompilerParams(use_tc_tiling_on_sc=False),
    out_type=jax.ShapeDtypeStruct((NW, CHUNKS, K, D), jnp.float32),
    scratch_types=[
        pltpu.VMEM((CHUNKS, K), jnp.int32),
        pltpu.VMEM((K, D), jnp.float32),
        pltpu.VMEM((K, D), jnp.float32),
        pltpu.SemaphoreType.DMA,
        pltpu.SemaphoreType.DMA,
    ],
)
def _embed_gather(idx_hbm, table_hbm, out_hbm, idx_v, buf0, buf1, sem0, sem1):
    wid = lax.axis_index("s") * NC + lax.axis_index("c")
    pltpu.sync_copy(idx_hbm.at[wid], idx_v)

    def fire(j, buf, sem):
        pltpu.async_copy(table_hbm.at[idx_v.at[j]], buf, sem)

    def drain(buf, sem):
        pltpu.make_async_copy(table_hbm.at[idx_v.at[0]], buf, sem).wait()

    fire(0, buf0, sem0)
    fire(1, buf1, sem1)

    def body(i, carry):
        j = 2 * i
        drain(buf0, sem0)
        pltpu.sync_copy(buf0, out_hbm.at[wid, j])
        fire(j + 2, buf0, sem0)
        drain(buf1, sem1)
        pltpu.sync_copy(buf1, out_hbm.at[wid, j + 1])
        fire(j + 3, buf1, sem1)
        return carry

    lax.fori_loop(0, (CHUNKS - 2) // 2, body, 0)
    drain(buf0, sem0)
    pltpu.sync_copy(buf0, out_hbm.at[wid, CHUNKS - 2])
    drain(buf1, sem1)
    pltpu.sync_copy(buf1, out_hbm.at[wid, CHUNKS - 1])


def kernel(inputs, embedding):
    n, m = inputs.shape
    idx = inputs.reshape(NW, CHUNKS, K).astype(jnp.int32)
    out = _embed_gather(idx, embedding)
    return out.reshape(n, m, D)


# speedup vs baseline: 1.0270x; 1.0270x over previous
import functools

import jax
import jax.numpy as jnp
from jax import lax
from jax.experimental import pallas as pl
from jax.experimental.pallas import tpu as pltpu
from jax.experimental.pallas import tpu_sc as plsc

NC = 2
NS = 16
NW = NC * NS
K = 128
D = 32
CHUNKS = 100
SUB = 5
NBUF = 4
KBIG = SUB * K
QCHUNKS = CHUNKS // SUB
NGROUPS = QCHUNKS // NBUF

_mesh = plsc.VectorSubcoreMesh(core_axis_name="c", subcore_axis_name="s")


@functools.partial(
    pl.kernel,
    mesh=_mesh,
    compiler_params=pltpu.CompilerParams(use_tc_tiling_on_sc=False),
    out_type=jax.ShapeDtypeStruct((NW, QCHUNKS, KBIG, D), jnp.float32),
    scratch_types=[
        pltpu.VMEM((CHUNKS, K), jnp.int32),
    ]
    + [pltpu.VMEM((KBIG, D), jnp.float32) for _ in range(NBUF)]
    + [pltpu.SemaphoreType.DMA for _ in range(2 * NBUF)],
)
def _embed_gather(idx_hbm, table_hbm, out_hbm, idx_v, *rest):
    bufs = rest[:NBUF]
    gsems = rest[NBUF:2 * NBUF]
    osems = rest[2 * NBUF:]
    wid = lax.axis_index("s") * NC + lax.axis_index("c")
    pltpu.sync_copy(idx_hbm.at[wid], idx_v)

    def fire_big(q, buf, gsem):
        for s in range(SUB):
            pltpu.async_copy(
                table_hbm.at[idx_v.at[q * SUB + s]],
                buf.at[pl.ds(s * K, K)],
                gsem,
            )

    def drain_big(buf, gsem):
        pltpu.make_async_copy(out_hbm.at[wid, 0], buf, gsem).wait()

    def fire_out(q, buf, osem):
        pltpu.async_copy(buf, out_hbm.at[wid, q], osem)

    def drain_out(buf, osem):
        pltpu.make_async_copy(buf, out_hbm.at[wid, 0], osem).wait()

    for b in range(NBUF):
        fire_big(b, bufs[b], gsems[b])

    def body(g, carry):
        for b in range(NBUF):
            drain_big(bufs[b], gsems[b])
            fire_out(g * NBUF + b, bufs[b], osems[b])
        for b in range(NBUF):
            drain_out(bufs[b], osems[b])
            fire_big((g + 1) * NBUF + b, bufs[b], gsems[b])
        return carry

    lax.fori_loop(0, NGROUPS - 1, body, 0)

    for b in range(NBUF):
        drain_big(bufs[b], gsems[b])
        fire_out((NGROUPS - 1) * NBUF + b, bufs[b], osems[b])
    for b in range(NBUF):
        drain_out(bufs[b], osems[b])


def kernel(inputs, embedding):
    n, m = inputs.shape
    idx = inputs.reshape(NW, CHUNKS, K).astype(jnp.int32)
    out = _embed_gather(idx, embedding)
    return out.reshape(n, m, D)


# speedup vs baseline: 1.3008x; 1.2666x over previous
import functools

import jax
import jax.numpy as jnp
from jax import lax
from jax.experimental import pallas as pl
from jax.experimental.pallas import tpu as pltpu
from jax.experimental.pallas import tpu_sc as plsc

NC = 2
NS = 16
NW = NC * NS
K = 128
D = 32
CHUNKS = 100
SUB = 5
NBUF = 4
KBIG = SUB * K
QCHUNKS = CHUNKS // SUB
NGROUPS = QCHUNKS // NBUF

V = 1000000
C_T = 4096
Q_T = C_T // 4
G_T = -(-V // C_T)
RPAD = G_T * Q_T

_mesh = plsc.VectorSubcoreMesh(core_axis_name="c", subcore_axis_name="s")


def _transpose_body(x_ref, o_ref):
    y = jnp.transpose(x_ref[...])
    o_ref[...] = jnp.concatenate(
        [y[0:Q_T], y[Q_T:2 * Q_T], y[2 * Q_T:3 * Q_T], y[3 * Q_T:]], axis=1
    )


def _tc_transpose(table_fm):
    return pl.pallas_call(
        _transpose_body,
        grid=(G_T,),
        in_specs=[pl.BlockSpec((32, C_T), lambda g: (0, g))],
        out_specs=pl.BlockSpec((Q_T, 128), lambda g: (g, 0)),
        out_shape=jax.ShapeDtypeStruct((RPAD, 128), jnp.float32),
    )(table_fm)


@functools.partial(
    pl.kernel,
    mesh=_mesh,
    compiler_params=pltpu.CompilerParams(use_tc_tiling_on_sc=False),
    out_type=jax.ShapeDtypeStruct((NW, QCHUNKS, KBIG, D), jnp.float32),
    scratch_types=[
        pltpu.VMEM((CHUNKS, K), jnp.int32),
    ]
    + [pltpu.VMEM((KBIG, D), jnp.float32) for _ in range(NBUF)]
    + [pltpu.SemaphoreType.DMA for _ in range(2 * NBUF)],
)
def _embed_gather(idx_hbm, table_hbm, out_hbm, idx_v, *rest):
    bufs = rest[:NBUF]
    gsems = rest[NBUF:2 * NBUF]
    osems = rest[2 * NBUF:]
    wid = lax.axis_index("s") * NC + lax.axis_index("c")
    pltpu.sync_copy(idx_hbm.at[wid], idx_v)

    def xform(m, carry):
        for c in range(K // 16):
            sl = pl.ds(c * 16, 16)
            v = idx_v[m, sl]
            j = (v & -4096) + ((v & 1023) << 2) + ((v >> 10) & 3)
            idx_v[m, sl] = j
        return carry

    lax.fori_loop(0, CHUNKS, xform, 0)

    def fire_big(q, buf, gsem):
        for s in range(SUB):
            pltpu.async_copy(
                table_hbm.at[idx_v.at[q * SUB + s]],
                buf.at[pl.ds(s * K, K)],
                gsem,
            )

    def drain_big(buf, gsem):
        pltpu.make_async_copy(out_hbm.at[wid, 0], buf, gsem).wait()

    def fire_out(q, buf, osem):
        pltpu.async_copy(buf, out_hbm.at[wid, q], osem)

    def drain_out(buf, osem):
        pltpu.make_async_copy(buf, out_hbm.at[wid, 0], osem).wait()

    for b in range(NBUF):
        fire_big(b, bufs[b], gsems[b])

    def body(g, carry):
        for b in range(NBUF):
            drain_big(bufs[b], gsems[b])
            fire_out(g * NBUF + b, bufs[b], osems[b])
        for b in range(NBUF):
            drain_out(bufs[b], osems[b])
            fire_big((g + 1) * NBUF + b, bufs[b], gsems[b])
        return carry

    lax.fori_loop(0, NGROUPS - 1, body, 0)

    for b in range(NBUF):
        drain_big(bufs[b], gsems[b])
        fire_out((NGROUPS - 1) * NBUF + b, bufs[b], osems[b])
    for b in range(NBUF):
        drain_out(bufs[b], osems[b])


def kernel(inputs, embedding):
    n, m = inputs.shape
    t4 = _tc_transpose(embedding.T)
    table = t4.reshape(RPAD * 4, D)
    idx = inputs.reshape(NW, CHUNKS, K).astype(jnp.int32)
    out = _embed_gather(idx, table)
    return out.reshape(n, m, D)


# speedup vs baseline: 1.3836x; 1.0636x over previous
import functools

import jax
import jax.numpy as jnp
from jax import lax
from jax.experimental import pallas as pl
from jax.experimental.pallas import tpu as pltpu
from jax.experimental.pallas import tpu_sc as plsc

NC = 2
NS = 16
NW = NC * NS
K = 128
D = 32
CHUNKS = 100
SUB = 5
NBUF = 4
KBIG = SUB * K
QCHUNKS = CHUNKS // SUB
NGROUPS = QCHUNKS // NBUF

V = 1000000
C_T = 4096
Q_T = C_T // 4
G_T = -(-V // C_T)
RPAD = G_T * Q_T

_mesh = plsc.VectorSubcoreMesh(core_axis_name="c", subcore_axis_name="s")


def _transpose_body(x_ref, o_ref):
    x = x_ref[...]
    in4 = jnp.concatenate(
        [x[:, a * Q_T:(a + 1) * Q_T] for a in range(4)], axis=0
    )
    o_ref[...] = jnp.transpose(in4)


def _tc_transpose(table_fm):
    return pl.pallas_call(
        _transpose_body,
        grid=(G_T,),
        in_specs=[pl.BlockSpec((32, C_T), lambda g: (0, g))],
        out_specs=pl.BlockSpec((Q_T, 128), lambda g: (g, 0)),
        out_shape=jax.ShapeDtypeStruct((RPAD, 128), jnp.float32),
    )(table_fm)


@functools.partial(
    pl.kernel,
    mesh=_mesh,
    compiler_params=pltpu.CompilerParams(use_tc_tiling_on_sc=False),
    out_type=jax.ShapeDtypeStruct((NW * CHUNKS * K, D), jnp.float32),
    scratch_types=[
        pltpu.VMEM((CHUNKS, K), jnp.int32),
    ]
    + [pltpu.VMEM((KBIG, D), jnp.float32) for _ in range(NBUF)]
    + [pltpu.SemaphoreType.DMA for _ in range(2 * NBUF)],
)
def _embed_gather(idx_hbm, table_hbm, out_hbm, idx_v, *rest):
    bufs = rest[:NBUF]
    gsems = rest[NBUF:2 * NBUF]
    osems = rest[2 * NBUF:]
    wid = lax.axis_index("s") * NC + lax.axis_index("c")
    pltpu.sync_copy(idx_hbm.at[wid], idx_v)

    def xform(m, carry):
        for c in range(K // 16):
            sl = pl.ds(c * 16, 16)
            v = idx_v[m, sl]
            j = (v & -4096) + ((v & 1023) << 2) + ((v >> 10) & 3)
            idx_v[m, sl] = j
        return carry

    lax.fori_loop(0, CHUNKS, xform, 0)

    def fire_big(q, buf, gsem):
        for s in range(SUB):
            pltpu.async_copy(
                table_hbm.at[idx_v.at[q * SUB + s]],
                buf.at[pl.ds(s * K, K)],
                gsem,
            )

    def drain_big(buf, gsem):
        pltpu.make_async_copy(out_hbm.at[pl.ds(0, KBIG)], buf, gsem).wait()

    def fire_out(q, buf, osem):
        base = wid * (CHUNKS * K) + q * KBIG
        pltpu.async_copy(buf, out_hbm.at[pl.ds(base, KBIG)], osem)

    def drain_out(buf, osem):
        pltpu.make_async_copy(buf, out_hbm.at[pl.ds(0, KBIG)], osem).wait()

    for b in range(NBUF):
        fire_big(b, bufs[b], gsems[b])

    def body(g, carry):
        for b in range(NBUF):
            drain_big(bufs[b], gsems[b])
            fire_out(g * NBUF + b, bufs[b], osems[b])
        for b in range(NBUF):
            drain_out(bufs[b], osems[b])
            fire_big((g + 1) * NBUF + b, bufs[b], gsems[b])
        return carry

    lax.fori_loop(0, NGROUPS - 1, body, 0)

    for b in range(NBUF):
        drain_big(bufs[b], gsems[b])
        fire_out((NGROUPS - 1) * NBUF + b, bufs[b], osems[b])
    for b in range(NBUF):
        drain_out(bufs[b], osems[b])


def kernel(inputs, embedding):
    n, m = inputs.shape
    t4 = _tc_transpose(embedding.T)
    table = t4.reshape(RPAD * 4, D)
    idx = inputs.reshape(NW, CHUNKS, K).astype(jnp.int32)
    out = _embed_gather(idx, table)
    return out.reshape(n, m, D)


# speedup vs baseline: 2.9304x; 2.1179x over previous
import functools

import jax
import jax.numpy as jnp
from jax import lax
from jax.experimental import pallas as pl
from jax.experimental.pallas import tpu as pltpu
from jax.experimental.pallas import tpu_sc as plsc

NC = 2
NS = 16
NW = NC * NS
K = 128
D = 32
CHUNKS = 100
NBUF = 10
NGROUPS = CHUNKS // NBUF
N = 4096
M = 100

V = 1000000
C_T = 4096
Q_T = C_T // 4
G_T = -(-V // C_T)
RPAD = G_T * Q_T

_mesh = plsc.VectorSubcoreMesh(core_axis_name="c", subcore_axis_name="s")


def _pack_body(x_ref, o_ref):
    x = x_ref[...]
    in4 = jnp.concatenate(
        [x[:, a * Q_T:(a + 1) * Q_T] for a in range(4)], axis=0
    )
    o_ref[...] = jnp.transpose(in4)


def _tc_pack(table_fm):
    return pl.pallas_call(
        _pack_body,
        grid=(G_T,),
        in_specs=[pl.BlockSpec((32, C_T), lambda g: (0, g))],
        out_specs=pl.BlockSpec((Q_T, 128), lambda g: (g, 0)),
        out_shape=jax.ShapeDtypeStruct((RPAD, 128), jnp.float32),
    )(table_fm)


def _unpack_body(x_ref, o_ref):
    x = x_ref[...]
    p = jnp.concatenate(
        [
            jnp.concatenate(
                [x[1024 * dm:1024 * (dm + 1), 32 * j:32 * (j + 1)]
                 for dm in range(4)],
                axis=1,
            )
            for j in range(4)
        ],
        axis=0,
    )
    o_ref[...] = jnp.transpose(p)


def _tc_unpack(x):
    return pl.pallas_call(
        _unpack_body,
        grid=(M // 4,),
        in_specs=[pl.BlockSpec((4 * 1024, 128), lambda g: (g, 0))],
        out_specs=pl.BlockSpec((128, N), lambda g: (g, 0)),
        out_shape=jax.ShapeDtypeStruct((M * D, N), jnp.float32),
    )(x)


@functools.partial(
    pl.kernel,
    mesh=_mesh,
    compiler_params=pltpu.CompilerParams(use_tc_tiling_on_sc=False),
    out_type=jax.ShapeDtypeStruct((M, N, D), jnp.float32),
    scratch_types=[
        pltpu.VMEM((CHUNKS, K), jnp.int32),
    ]
    + [pltpu.VMEM((K, D), jnp.float32) for _ in range(NBUF)]
    + [pltpu.SemaphoreType.DMA for _ in range(2 * NBUF)],
)
def _embed_gather(idx_hbm, table_hbm, out_hbm, idx_v, *rest):
    bufs = rest[:NBUF]
    gsems = rest[NBUF:2 * NBUF]
    osems = rest[2 * NBUF:]
    wid = lax.axis_index("s") * NC + lax.axis_index("c")
    pbase = wid * K
    pltpu.sync_copy(idx_hbm.at[:, pl.ds(pbase, K)], idx_v)

    def xform(m, carry):
        for c in range(K // 16):
            sl = pl.ds(c * 16, 16)
            v = idx_v[m, sl]
            j = (v & -4096) + ((v & 1023) << 2) + ((v >> 10) & 3)
            idx_v[m, sl] = j
        return carry

    lax.fori_loop(0, CHUNKS, xform, 0)

    def fire_gather(m, b):
        pltpu.async_copy(table_hbm.at[idx_v.at[m]], bufs[b], gsems[b])

    def drain_gather(b):
        pltpu.make_async_copy(
            table_hbm.at[idx_v.at[0]], bufs[b], gsems[b]
        ).wait()

    def fire_out(m, b):
        pltpu.async_copy(bufs[b], out_hbm.at[m, pl.ds(pbase, K)], osems[b])

    def drain_out(b):
        pltpu.make_async_copy(
            bufs[b], out_hbm.at[0, pl.ds(0, K)], osems[b]
        ).wait()

    for b in range(NBUF):
        fire_gather(b, b)

    def body(g, carry):
        for b in range(NBUF):
            drain_gather(b)
            fire_out(g * NBUF + b, b)
        for b in range(NBUF):
            drain_out(b)
            fire_gather((g + 1) * NBUF + b, b)
        return carry

    lax.fori_loop(0, NGROUPS - 1, body, 0)

    for b in range(NBUF):
        m = (NGROUPS - 1) * NBUF + b
        drain_gather(b)
        fire_out(m, b)
    for b in range(NBUF):
        drain_out(b)


def kernel(inputs, embedding):
    t4 = _tc_pack(embedding.T)
    table = t4.reshape(RPAD * 4, D)
    idx_p = (
        inputs.T.astype(jnp.int32)
        .reshape(M, 4, N // 4)
        .transpose(0, 2, 1)
        .reshape(M, N)
    )
    out_sc = _embed_gather(idx_p, table)
    out4 = _tc_unpack(out_sc.reshape(M * N * D // 128, 128))
    return jnp.transpose(out4.reshape(M, D, N), (2, 0, 1))


# speedup vs baseline: 4.5078x; 1.5383x over previous
import functools

import jax
import jax.numpy as jnp
from jax import lax
from jax.experimental import pallas as pl
from jax.experimental.pallas import tpu as pltpu
from jax.experimental.pallas import tpu_sc as plsc

NC = 2
NS = 16
NW = NC * NS
K = 128
D = 32
CHUNKS = 100
NBUF = 10
NGROUPS = CHUNKS // NBUF
N = 4096
M = 100

V = 1000000
C_T = 16384
Q_T = C_T // 4
G_T = -(-V // C_T)
RPAD = G_T * Q_T

_mesh = plsc.VectorSubcoreMesh(core_axis_name="c", subcore_axis_name="s")


def _pack_body(x_ref, o_ref):
    x = x_ref[...]
    in4 = jnp.concatenate(
        [x[:, a * Q_T:(a + 1) * Q_T] for a in range(4)], axis=0
    )
    o_ref[...] = jnp.transpose(in4)


def _tc_pack(table_fm):
    return pl.pallas_call(
        _pack_body,
        grid=(G_T,),
        in_specs=[pl.BlockSpec((32, C_T), lambda g: (0, g))],
        out_specs=pl.BlockSpec((Q_T, 128), lambda g: (g, 0)),
        out_shape=jax.ShapeDtypeStruct((RPAD, 128), jnp.float32),
    )(table_fm)


def _unpack_body(x_ref, o_ref):
    xt = jnp.transpose(x_ref[...])
    o_ref[...] = jnp.concatenate(
        [
            jnp.concatenate(
                [xt[32 * j:32 * (j + 1), 1024 * dm:1024 * (dm + 1)]
                 for j in range(4)],
                axis=1,
            )
            for dm in range(4)
        ],
        axis=0,
    )


def _tc_unpack(x):
    return pl.pallas_call(
        _unpack_body,
        grid=(M // 4,),
        in_specs=[pl.BlockSpec((4 * 1024, 128), lambda g: (g, 0))],
        out_specs=pl.BlockSpec((128, N), lambda g: (g, 0)),
        out_shape=jax.ShapeDtypeStruct((M * D, N), jnp.float32),
    )(x)


@functools.partial(
    pl.kernel,
    mesh=_mesh,
    compiler_params=pltpu.CompilerParams(use_tc_tiling_on_sc=False),
    out_type=jax.ShapeDtypeStruct((M, N, D), jnp.float32),
    scratch_types=[
        pltpu.VMEM((CHUNKS, K), jnp.int32),
    ]
    + [pltpu.VMEM((K, D), jnp.float32) for _ in range(NBUF)]
    + [pltpu.SemaphoreType.DMA for _ in range(2 * NBUF)],
)
def _embed_gather(idx_hbm, table_hbm, out_hbm, idx_v, *rest):
    bufs = rest[:NBUF]
    gsems = rest[NBUF:2 * NBUF]
    osems = rest[2 * NBUF:]
    wid = lax.axis_index("s") * NC + lax.axis_index("c")
    pbase = wid * K
    pltpu.sync_copy(idx_hbm.at[:, pl.ds(pbase, K)], idx_v)

    def xform(m, carry):
        for c in range(K // 16):
            sl = pl.ds(c * 16, 16)
            v = idx_v[m, sl]
            j = (v & -C_T) + ((v & (Q_T - 1)) << 2) + ((v >> 12) & 3)
            idx_v[m, sl] = j
        return carry

    lax.fori_loop(0, CHUNKS, xform, 0)

    def fire_gather(m, b):
        pltpu.async_copy(table_hbm.at[idx_v.at[m]], bufs[b], gsems[b])

    def drain_gather(b):
        pltpu.make_async_copy(
            table_hbm.at[idx_v.at[0]], bufs[b], gsems[b]
        ).wait()

    def fire_out(m, b):
        pltpu.async_copy(bufs[b], out_hbm.at[m, pl.ds(pbase, K)], osems[b])

    def drain_out(b):
        pltpu.make_async_copy(
            bufs[b], out_hbm.at[0, pl.ds(0, K)], osems[b]
        ).wait()

    for b in range(NBUF):
        fire_gather(b, b)

    def body(g, carry):
        for b in range(NBUF):
            drain_gather(b)
            fire_out(g * NBUF + b, b)
        for b in range(NBUF):
            drain_out(b)
            fire_gather((g + 1) * NBUF + b, b)
        return carry

    lax.fori_loop(0, NGROUPS - 1, body, 0)

    for b in range(NBUF):
        m = (NGROUPS - 1) * NBUF + b
        drain_gather(b)
        fire_out(m, b)
    for b in range(NBUF):
        drain_out(b)


def kernel(inputs, embedding):
    t4 = _tc_pack(embedding.T)
    table = t4.reshape(RPAD * 4, D)
    idx_p = (
        inputs.T.astype(jnp.int32)
        .reshape(M, 4, N // 4)
        .transpose(0, 2, 1)
        .reshape(M, N)
    )
    out_sc = _embed_gather(idx_p, table)
    out4 = _tc_unpack(out_sc.reshape(M * N * D // 128, 128))
    return jnp.transpose(out4.reshape(M, D, N), (2, 0, 1))
